# baseline (speedup 1.0000x reference)
import math

import jax
import jax.numpy as jnp
from jax.experimental import pallas as pl
from jax.experimental.pallas import tpu as pltpu

B = 8; T = 1500; W = 128; NH = 2; HD = 64; CD = 2; KC = 513; FF = 512
SCALE = 1.0 / math.sqrt(HD)


def _ln(x, g, b):
    m = jnp.mean(x, axis=-1, keepdims=True)
    xc = x - m
    v = jnp.mean(xc * xc, axis=-1, keepdims=True)
    return xc * jax.lax.rsqrt(v + 1e-5) * g + b


def _gelu(x):
    return 0.5 * x * (1.0 + jnp.tanh(jnp.sqrt(2.0 / jnp.pi) * (x + 0.044715 * x ** 3)))


def _fused_kernel(embs_ref, mask_ref, mlg_ref, mlb_ref, mw1_ref, mb1_ref,
                  mw2_ref, mb2_ref, piw_ref, pib_ref, pow_ref, pob_ref,
                  cb_ref, cbt_ref, pos_ref, l1g_ref, l1b_ref, wq_ref, wk_ref,
                  wv_ref, wo_ref, l2g_ref, l2b_ref, fw1_ref, fb1_ref,
                  fw2_ref, fb2_ref, lpg_ref, lpb_ref,
                  out_ref, idx_ref, commit_ref):
    f32 = jnp.float32
    x = embs_ref[0]

    h = _ln(x, mlg_ref[:], mlb_ref[:])
    h1 = _gelu(jnp.dot(h, mw1_ref[:], preferred_element_type=f32) + mb1_ref[:])
    x = x + jnp.dot(h1, mw2_ref[:], preferred_element_type=f32) + mb2_ref[:]

    z3 = jnp.dot(x, piw_ref[:], preferred_element_type=f32) + pib_ref[:]
    z = z3[:, :CD]
    sc = jnp.dot(z3, cbt_ref[:], preferred_element_type=f32)
    smax = jnp.max(sc, axis=-1, keepdims=True)
    ids = jax.lax.broadcasted_iota(jnp.int32, sc.shape, 1)
    idx = jnp.min(jnp.where(sc == smax, ids, KC), axis=-1, keepdims=True)
    idx_ref[0] = idx
    onehot = (ids == idx).astype(f32)
    q = jnp.dot(onehot, cb_ref[:], preferred_element_type=f32)
    diff = q - z
    commit_ref[0] = jnp.full((1, W), jnp.sum(diff * diff), dtype=f32)
    qst = z + (q - z)
    quant = jnp.dot(qst, pow_ref[:], preferred_element_type=f32) + pob_ref[:]

    mvec = jnp.dot(cb_ref[KC - 1:KC, :], pow_ref[:],
                   preferred_element_type=f32) + pob_ref[:]
    m = mask_ref[0]
    x = jnp.where(m > 0, quant, mvec) + pos_ref[:]

    h = _ln(x, l1g_ref[:], l1b_ref[:])
    qm = jnp.dot(h, wq_ref[:], preferred_element_type=f32)
    km = jnp.dot(h, wk_ref[:], preferred_element_type=f32)
    vm = jnp.dot(h, wv_ref[:], preferred_element_type=f32)
    bf16 = jnp.bfloat16
    ones_col = jnp.ones((x.shape[0], 1), dtype=bf16)
    outs = []
    for n in range(NH):
        sl = slice(n * HD, (n + 1) * HD)
        s = jax.lax.dot_general(qm[:, sl], km[:, sl],
                                (((1,), (1,)), ((), ())),
                                preferred_element_type=f32)
        e = jnp.exp2(s).astype(bf16)
        ve = jnp.concatenate([vm[:, sl].astype(bf16), ones_col], axis=1)
        oe = jnp.dot(e, ve, preferred_element_type=f32)
        r = 1.0 / oe[:, HD:HD + 1]
        outs.append(oe[:, :HD] * r)
    o = jnp.concatenate(outs, axis=1)
    x = x + jnp.dot(o, wo_ref[:], preferred_element_type=f32)

    h = _ln(x, l2g_ref[:], l2b_ref[:])
    h1 = _gelu(jnp.dot(h.astype(bf16), fw1_ref[:],
                       preferred_element_type=f32) + fb1_ref[:])
    x = x + jnp.dot(h1.astype(bf16), fw2_ref[:],
                    preferred_element_type=f32) + fb2_ref[:]

    out_ref[0] = _ln(x, lpg_ref[:], lpb_ref[:])


def kernel(embs, mask, mlp_ln_g, mlp_ln_b, mlp_w1, mlp_b1, mlp_w2, mlp_b2,
           proj_in_w, proj_in_b, proj_out_w, proj_out_b, codebook, pos_emb,
           ln1_g, ln1_b, wq, wk, wv, wo, ln2_g, ln2_b,
           ffn_w1, ffn_b1, ffn_w2, ffn_b2, lnp_g, lnp_b):
    mask3 = mask.astype(jnp.float32).reshape(B, T, 1)
    row = lambda v: v.reshape(1, -1)
    piw3 = jnp.concatenate([proj_in_w, jnp.zeros((W, 1), jnp.float32)], axis=1)
    pib3 = jnp.concatenate([proj_in_b, jnp.ones((1,), jnp.float32)]).reshape(1, CD + 1)
    cbt3 = jnp.concatenate(
        [codebook.T, -0.5 * jnp.sum(codebook * codebook, axis=1)[None, :]], axis=0)
    wq_s = wq * (SCALE * math.log2(math.e))

    def full(shape):
        zeros = (0,) * len(shape)
        return pl.BlockSpec(shape, lambda b: zeros)

    in_specs = [
        pl.BlockSpec((1, T, W), lambda b: (b, 0, 0)),
        pl.BlockSpec((1, T, 1), lambda b: (b, 0, 0)),
        full((1, W)), full((1, W)),
        full((W, FF)), full((1, FF)),
        full((FF, W)), full((1, W)),
        full((W, CD + 1)), full((1, CD + 1)),
        full((CD, W)), full((1, W)),
        full((KC, CD)), full((CD + 1, KC)),
        full((T, W)),
        full((1, W)), full((1, W)),
        full((W, W)), full((W, W)), full((W, W)), full((W, W)),
        full((1, W)), full((1, W)),
        full((W, FF)), full((1, FF)),
        full((FF, W)), full((1, W)),
        full((1, W)), full((1, W)),
    ]
    out_specs = [
        pl.BlockSpec((1, T, W), lambda b: (b, 0, 0)),
        pl.BlockSpec((1, T, 1), lambda b: (b, 0, 0)),
        pl.BlockSpec((1, 1, W), lambda b: (b, 0, 0)),
    ]
    out, idx3, commit = pl.pallas_call(
        _fused_kernel,
        grid=(B,),
        in_specs=in_specs,
        out_specs=out_specs,
        out_shape=[
            jax.ShapeDtypeStruct((B, T, W), jnp.float32),
            jax.ShapeDtypeStruct((B, T, 1), jnp.int32),
            jax.ShapeDtypeStruct((B, 1, W), jnp.float32),
        ],
        compiler_params=pltpu.CompilerParams(
            dimension_semantics=("parallel",)),
    )(embs, mask3, row(mlp_ln_g), row(mlp_ln_b), mlp_w1, row(mlp_b1),
      mlp_w2, row(mlp_b2), piw3, pib3, proj_out_w,
      row(proj_out_b), codebook, cbt3, pos_emb, row(ln1_g), row(ln1_b),
      wq_s, wk, wv, wo, row(ln2_g), row(ln2_b),
      ffn_w1.astype(jnp.bfloat16), row(ffn_b1),
      ffn_w2.astype(jnp.bfloat16), row(ffn_b2), row(lnp_g), row(lnp_b))
    idx = idx3.reshape(B, T)
    commit_loss = jnp.sum(commit[:, 0, 0]) / (B * T * CD)
    return out, idx, commit_loss


# speedup vs baseline: 1.0110x; 1.0110x over previous
import math

import jax
import jax.numpy as jnp
from jax.experimental import pallas as pl
from jax.experimental.pallas import tpu as pltpu

B = 8; T = 1500; W = 128; NH = 2; HD = 64; CD = 2; KC = 513; FF = 512
SCALE = 1.0 / math.sqrt(HD)


def _ln(x, g, b):
    m = jnp.mean(x, axis=-1, keepdims=True)
    xc = x - m
    v = jnp.mean(xc * xc, axis=-1, keepdims=True)
    return xc * jax.lax.rsqrt(v + 1e-5) * g + b


def _gelu(x):
    return 0.5 * x * (1.0 + jnp.tanh(jnp.sqrt(2.0 / jnp.pi) * (x + 0.044715 * x ** 3)))


def _fused_kernel(embs_ref, mask_ref, mlg_ref, mlb_ref, mw1_ref, mb1_ref,
                  mw2_ref, mb2_ref, piw_ref, pib_ref, pow_ref, pob_ref,
                  cb_ref, cbt_ref, pos_ref, l1g_ref, l1b_ref, wq_ref, wk_ref,
                  wv_ref, wo_ref, l2g_ref, l2b_ref, fw1_ref, fb1_ref,
                  fw2_ref, fb2_ref, lpg_ref, lpb_ref,
                  out_ref, idx_ref, commit_ref):
    f32 = jnp.float32
    x = embs_ref[0]

    h = _ln(x, mlg_ref[:], mlb_ref[:])
    h1 = _gelu(jnp.dot(h, mw1_ref[:], preferred_element_type=f32) + mb1_ref[:])
    x = x + jnp.dot(h1, mw2_ref[:], preferred_element_type=f32) + mb2_ref[:]

    z = jnp.dot(x, piw_ref[:], preferred_element_type=f32) + pib_ref[:]
    zsq = jnp.sum(z * z, axis=-1, keepdims=True)
    cbt = cbt_ref[:]
    csq = jnp.sum(cbt * cbt, axis=0, keepdims=True)
    d = zsq - 2.0 * jnp.dot(z, cbt, preferred_element_type=f32) + csq
    dmin = jnp.min(d, axis=-1, keepdims=True)
    ids = jax.lax.broadcasted_iota(jnp.int32, d.shape, 1)
    idx = jnp.min(jnp.where(d == dmin, ids, KC), axis=-1, keepdims=True)
    idx_ref[0] = idx
    onehot = (ids == idx).astype(f32)
    q = jnp.dot(onehot, cb_ref[:], preferred_element_type=f32)
    diff = q - z
    commit_ref[0] = jnp.full((1, W), jnp.sum(diff * diff), dtype=f32)
    qst = z + (q - z)
    quant = jnp.dot(qst, pow_ref[:], preferred_element_type=f32) + pob_ref[:]

    mvec = jnp.dot(cb_ref[KC - 1:KC, :], pow_ref[:],
                   preferred_element_type=f32) + pob_ref[:]
    m = mask_ref[0]
    x = jnp.where(m > 0, quant, mvec) + pos_ref[:]

    h = _ln(x, l1g_ref[:], l1b_ref[:])
    qm = jnp.dot(h, wq_ref[:], preferred_element_type=f32)
    km = jnp.dot(h, wk_ref[:], preferred_element_type=f32)
    vm = jnp.dot(h, wv_ref[:], preferred_element_type=f32)
    bf16 = jnp.bfloat16
    ones_col = jnp.ones((x.shape[0], 1), dtype=bf16)
    outs = []
    for n in range(NH):
        sl = slice(n * HD, (n + 1) * HD)
        s = jax.lax.dot_general(qm[:, sl], km[:, sl],
                                (((1,), (1,)), ((), ())),
                                preferred_element_type=f32)
        e = jnp.exp2(s).astype(bf16)
        ve = jnp.concatenate([vm[:, sl].astype(bf16), ones_col], axis=1)
        oe = jnp.dot(e, ve, preferred_element_type=f32)
        r = 1.0 / oe[:, HD:HD + 1]
        outs.append(oe[:, :HD] * r)
    o = jnp.concatenate(outs, axis=1)
    x = x + jnp.dot(o, wo_ref[:], preferred_element_type=f32)

    h = _ln(x, l2g_ref[:], l2b_ref[:])
    h1 = _gelu(jnp.dot(h.astype(bf16), fw1_ref[:],
                       preferred_element_type=f32) + fb1_ref[:])
    x = x + jnp.dot(h1.astype(bf16), fw2_ref[:],
                    preferred_element_type=f32) + fb2_ref[:]

    out_ref[0] = _ln(x, lpg_ref[:], lpb_ref[:])


def kernel(embs, mask, mlp_ln_g, mlp_ln_b, mlp_w1, mlp_b1, mlp_w2, mlp_b2,
           proj_in_w, proj_in_b, proj_out_w, proj_out_b, codebook, pos_emb,
           ln1_g, ln1_b, wq, wk, wv, wo, ln2_g, ln2_b,
           ffn_w1, ffn_b1, ffn_w2, ffn_b2, lnp_g, lnp_b):
    mask3 = mask.astype(jnp.float32).reshape(B, T, 1)
    row = lambda v: v.reshape(1, -1)
    wq_s = wq * (SCALE * math.log2(math.e))

    def full(shape):
        zeros = (0,) * len(shape)
        return pl.BlockSpec(shape, lambda b: zeros)

    in_specs = [
        pl.BlockSpec((1, T, W), lambda b: (b, 0, 0)),
        pl.BlockSpec((1, T, 1), lambda b: (b, 0, 0)),
        full((1, W)), full((1, W)),
        full((W, FF)), full((1, FF)),
        full((FF, W)), full((1, W)),
        full((W, CD)), full((1, CD)),
        full((CD, W)), full((1, W)),
        full((KC, CD)), full((CD, KC)),
        full((T, W)),
        full((1, W)), full((1, W)),
        full((W, W)), full((W, W)), full((W, W)), full((W, W)),
        full((1, W)), full((1, W)),
        full((W, FF)), full((1, FF)),
        full((FF, W)), full((1, W)),
        full((1, W)), full((1, W)),
    ]
    out_specs = [
        pl.BlockSpec((1, T, W), lambda b: (b, 0, 0)),
        pl.BlockSpec((1, T, 1), lambda b: (b, 0, 0)),
        pl.BlockSpec((1, 1, W), lambda b: (b, 0, 0)),
    ]
    out, idx3, commit = pl.pallas_call(
        _fused_kernel,
        grid=(B,),
        in_specs=in_specs,
        out_specs=out_specs,
        out_shape=[
            jax.ShapeDtypeStruct((B, T, W), jnp.float32),
            jax.ShapeDtypeStruct((B, T, 1), jnp.int32),
            jax.ShapeDtypeStruct((B, 1, W), jnp.float32),
        ],
        compiler_params=pltpu.CompilerParams(
            dimension_semantics=("parallel",)),
    )(embs, mask3, row(mlp_ln_g), row(mlp_ln_b), mlp_w1, row(mlp_b1),
      mlp_w2, row(mlp_b2), proj_in_w, row(proj_in_b), proj_out_w,
      row(proj_out_b), codebook, codebook.T, pos_emb, row(ln1_g), row(ln1_b),
      wq_s, wk, wv, wo, row(ln2_g), row(ln2_b),
      ffn_w1.astype(jnp.bfloat16), row(ffn_b1),
      ffn_w2.astype(jnp.bfloat16), row(ffn_b2), row(lnp_g), row(lnp_b))
    idx = idx3.reshape(B, T)
    commit_loss = jnp.sum(commit[:, 0, 0]) / (B * T * CD)
    return out, idx, commit_loss


# speedup vs baseline: 1.0521x; 1.0407x over previous
import math

import jax
import jax.numpy as jnp
from jax.experimental import pallas as pl
from jax.experimental.pallas import tpu as pltpu

B = 8; T = 1500; W = 128; NH = 2; HD = 64; CD = 2; KC = 513; FF = 512
SCALE = 1.0 / math.sqrt(HD)


def _ln(x, g, b):
    m = jnp.mean(x, axis=-1, keepdims=True)
    xc = x - m
    v = jnp.mean(xc * xc, axis=-1, keepdims=True)
    return xc * jax.lax.rsqrt(v + 1e-5) * g + b


def _gelu(x):
    return 0.5 * x * (1.0 + jnp.tanh(jnp.sqrt(2.0 / jnp.pi) * (x + 0.044715 * x ** 3)))


def _fused_kernel(embs_ref, mask_ref, mlg_ref, mlb_ref, mw1_ref, mb1_ref,
                  mw2_ref, mb2_ref, piw_ref, pib_ref, pow_ref, pob_ref,
                  cb_ref, cbt_ref, pos_ref, l1g_ref, l1b_ref, wq_ref, wk_ref,
                  wv_ref, wo_ref, l2g_ref, l2b_ref, fw1_ref, fb1_ref,
                  fw2_ref, fb2_ref, lpg_ref, lpb_ref,
                  out_ref, idx_ref, commit_ref):
    f32 = jnp.float32
    x = embs_ref[0]

    h = _ln(x, mlg_ref[:], mlb_ref[:])
    h1 = _gelu(jnp.dot(h, mw1_ref[:], preferred_element_type=f32) + mb1_ref[:])
    x = x + jnp.dot(h1, mw2_ref[:], preferred_element_type=f32) + mb2_ref[:]

    z = jnp.dot(x, piw_ref[:], preferred_element_type=f32) + pib_ref[:]
    zsq = jnp.sum(z * z, axis=-1, keepdims=True)
    cbt = cbt_ref[:]
    csq = jnp.sum(cbt * cbt, axis=0, keepdims=True)
    d = zsq - 2.0 * jnp.dot(z, cbt, preferred_element_type=f32) + csq
    dmin = jnp.min(d, axis=-1, keepdims=True)
    ids = jax.lax.broadcasted_iota(jnp.int32, d.shape, 1)
    idx = jnp.min(jnp.where(d == dmin, ids, KC), axis=-1, keepdims=True)
    idx_ref[0] = idx
    onehot = (ids == idx).astype(f32)
    q = jnp.dot(onehot, cb_ref[:], preferred_element_type=f32)
    diff = q - z
    commit_ref[0] = jnp.full((1, W), jnp.sum(diff * diff), dtype=f32)
    qst = z + (q - z)
    quant = jnp.dot(qst, pow_ref[:], preferred_element_type=f32) + pob_ref[:]

    mvec = jnp.dot(cb_ref[KC - 1:KC, :], pow_ref[:],
                   preferred_element_type=f32) + pob_ref[:]
    m = mask_ref[0]
    x = jnp.where(m > 0, quant, mvec) + pos_ref[:]

    h = _ln(x, l1g_ref[:], l1b_ref[:])
    qm = jnp.dot(h, wq_ref[:], preferred_element_type=f32)
    km = jnp.dot(h, wk_ref[:], preferred_element_type=f32)
    vm = jnp.dot(h, wv_ref[:], preferred_element_type=f32)
    ones_col = jnp.ones((x.shape[0], 1), dtype=f32)
    outs = []
    for n in range(NH):
        sl = slice(n * HD, (n + 1) * HD)
        s = jax.lax.dot_general(qm[:, sl], km[:, sl],
                                (((1,), (1,)), ((), ())),
                                preferred_element_type=f32)
        e = jnp.exp2(s)
        ve = jnp.concatenate([vm[:, sl], ones_col], axis=1)
        oe = jnp.dot(e, ve, preferred_element_type=f32)
        r = 1.0 / oe[:, HD:HD + 1]
        outs.append(oe[:, :HD] * r)
    o = jnp.concatenate(outs, axis=1)
    x = x + jnp.dot(o, wo_ref[:], preferred_element_type=f32)

    h = _ln(x, l2g_ref[:], l2b_ref[:])
    h1 = _gelu(jnp.dot(h, fw1_ref[:], preferred_element_type=f32) + fb1_ref[:])
    x = x + jnp.dot(h1, fw2_ref[:], preferred_element_type=f32) + fb2_ref[:]

    out_ref[0] = _ln(x, lpg_ref[:], lpb_ref[:])


def kernel(embs, mask, mlp_ln_g, mlp_ln_b, mlp_w1, mlp_b1, mlp_w2, mlp_b2,
           proj_in_w, proj_in_b, proj_out_w, proj_out_b, codebook, pos_emb,
           ln1_g, ln1_b, wq, wk, wv, wo, ln2_g, ln2_b,
           ffn_w1, ffn_b1, ffn_w2, ffn_b2, lnp_g, lnp_b):
    mask3 = mask.astype(jnp.float32).reshape(B, T, 1)
    row = lambda v: v.reshape(1, -1)
    wq_s = wq * (SCALE * math.log2(math.e))

    def full(shape):
        zeros = (0,) * len(shape)
        return pl.BlockSpec(shape, lambda b: zeros)

    in_specs = [
        pl.BlockSpec((1, T, W), lambda b: (b, 0, 0)),
        pl.BlockSpec((1, T, 1), lambda b: (b, 0, 0)),
        full((1, W)), full((1, W)),
        full((W, FF)), full((1, FF)),
        full((FF, W)), full((1, W)),
        full((W, CD)), full((1, CD)),
        full((CD, W)), full((1, W)),
        full((KC, CD)), full((CD, KC)),
        full((T, W)),
        full((1, W)), full((1, W)),
        full((W, W)), full((W, W)), full((W, W)), full((W, W)),
        full((1, W)), full((1, W)),
        full((W, FF)), full((1, FF)),
        full((FF, W)), full((1, W)),
        full((1, W)), full((1, W)),
    ]
    out_specs = [
        pl.BlockSpec((1, T, W), lambda b: (b, 0, 0)),
        pl.BlockSpec((1, T, 1), lambda b: (b, 0, 0)),
        pl.BlockSpec((1, 1, W), lambda b: (b, 0, 0)),
    ]
    out, idx3, commit = pl.pallas_call(
        _fused_kernel,
        grid=(B,),
        in_specs=in_specs,
        out_specs=out_specs,
        out_shape=[
            jax.ShapeDtypeStruct((B, T, W), jnp.float32),
            jax.ShapeDtypeStruct((B, T, 1), jnp.int32),
            jax.ShapeDtypeStruct((B, 1, W), jnp.float32),
        ],
        compiler_params=pltpu.CompilerParams(
            dimension_semantics=("parallel",)),
    )(embs, mask3, row(mlp_ln_g), row(mlp_ln_b), mlp_w1, row(mlp_b1),
      mlp_w2, row(mlp_b2), proj_in_w, row(proj_in_b), proj_out_w,
      row(proj_out_b), codebook, codebook.T, pos_emb, row(ln1_g), row(ln1_b),
      wq_s, wk, wv, wo, row(ln2_g), row(ln2_b),
      ffn_w1, row(ffn_b1),
      ffn_w2, row(ffn_b2), row(lnp_g), row(lnp_b))
    idx = idx3.reshape(B, T)
    commit_loss = jnp.sum(commit[:, 0, 0]) / (B * T * CD)
    return out, idx, commit_loss


# speedup vs baseline: 1.0887x; 1.0348x over previous
import math

import jax
import jax.numpy as jnp
from jax.experimental import pallas as pl
from jax.experimental.pallas import tpu as pltpu

B = 8; T = 1500; W = 128; NH = 2; HD = 64; CD = 2; KC = 513; FF = 512
SCALE = 1.0 / math.sqrt(HD)
_QSCALE = SCALE * math.log2(math.e)


def _ln(x, g, b):
    m = jnp.mean(x, axis=-1, keepdims=True)
    xc = x - m
    v = jnp.mean(xc * xc, axis=-1, keepdims=True)
    return xc * jax.lax.rsqrt(v + 1e-5) * g + b


_GC1 = math.sqrt(2.0 / math.pi)
_GC2 = _GC1 * 0.044715


def _gelu(x):
    u = x * x
    t = jnp.tanh(x * (_GC1 + _GC2 * u))
    xh = 0.5 * x
    return xh * t + xh


def _fused_kernel(embs_ref, mask_ref, mlg_ref, mlb_ref, mw1_ref, mb1_ref,
                  mw2_ref, mb2_ref, piw_ref, pib_ref, pow_ref, pob_ref,
                  cb_ref, pos_ref, l1g_ref, l1b_ref, wq_ref, wk_ref,
                  wv_ref, wo_ref, l2g_ref, l2b_ref, fw1_ref, fb1_ref,
                  fw2_ref, fb2_ref, lpg_ref, lpb_ref,
                  out_ref, idx_ref, commit_ref):
    f32 = jnp.float32
    x = embs_ref[0]

    h = _ln(x, mlg_ref[:], mlb_ref[:])
    h1 = _gelu(jnp.dot(h, mw1_ref[:], preferred_element_type=f32) + mb1_ref[:])
    x = x + jnp.dot(h1, mw2_ref[:], preferred_element_type=f32) + mb2_ref[:]

    cb = cb_ref[:]
    z = jnp.dot(x, piw_ref[:], preferred_element_type=f32) + pib_ref[:]
    zsq = jnp.sum(z * z, axis=-1, keepdims=True)
    csq = jnp.sum(cb * cb, axis=-1)[None, :]
    zc = jax.lax.dot_general(z, cb, (((1,), (1,)), ((), ())),
                             preferred_element_type=f32)
    d = zsq - 2.0 * zc + csq
    dmin = jnp.min(d, axis=-1, keepdims=True)
    ids = jax.lax.broadcasted_iota(jnp.int32, d.shape, 1)
    idx = jnp.min(jnp.where(d == dmin, ids, KC), axis=-1, keepdims=True)
    idx_ref[0] = idx
    onehot = (ids == idx).astype(f32)
    q = jnp.dot(onehot, cb, preferred_element_type=f32)
    diff = q - z
    commit_ref[0] = jnp.full((1, W), jnp.sum(diff * diff), dtype=f32)
    qst = z + (q - z)

    m = mask_ref[0]
    qmask = jnp.where(m > 0, qst, cb[KC - 1:KC, :])
    x = jnp.dot(qmask, pow_ref[:],
                preferred_element_type=f32) + pob_ref[:] + pos_ref[:]

    h = _ln(x, l1g_ref[:], l1b_ref[:])
    qm = jnp.dot(h, wq_ref[:], preferred_element_type=f32) * _QSCALE
    km = jnp.dot(h, wk_ref[:], preferred_element_type=f32)
    vm = jnp.dot(h, wv_ref[:], preferred_element_type=f32)
    ones_col = jnp.ones((x.shape[0], 1), dtype=f32)
    outs = []
    for n in range(NH):
        sl = slice(n * HD, (n + 1) * HD)
        s = jax.lax.dot_general(qm[:, sl], km[:, sl],
                                (((1,), (1,)), ((), ())),
                                preferred_element_type=f32)
        e = jnp.exp2(s)
        ve = jnp.concatenate([vm[:, sl], ones_col], axis=1)
        oe = jnp.dot(e, ve, preferred_element_type=f32)
        r = 1.0 / oe[:, HD:HD + 1]
        outs.append(oe[:, :HD] * r)
    o = jnp.concatenate(outs, axis=1)
    x = x + jnp.dot(o, wo_ref[:], preferred_element_type=f32)

    h = _ln(x, l2g_ref[:], l2b_ref[:])
    h1 = _gelu(jnp.dot(h, fw1_ref[:], preferred_element_type=f32) + fb1_ref[:])
    x = x + jnp.dot(h1, fw2_ref[:], preferred_element_type=f32) + fb2_ref[:]

    out_ref[0] = _ln(x, lpg_ref[:], lpb_ref[:])


def kernel(embs, mask, mlp_ln_g, mlp_ln_b, mlp_w1, mlp_b1, mlp_w2, mlp_b2,
           proj_in_w, proj_in_b, proj_out_w, proj_out_b, codebook, pos_emb,
           ln1_g, ln1_b, wq, wk, wv, wo, ln2_g, ln2_b,
           ffn_w1, ffn_b1, ffn_w2, ffn_b2, lnp_g, lnp_b):
    mask3 = mask.astype(jnp.float32).reshape(B, T, 1)
    row = lambda v: v.reshape(1, -1)

    def full(shape):
        zeros = (0,) * len(shape)
        return pl.BlockSpec(shape, lambda b: zeros)

    in_specs = [
        pl.BlockSpec((1, T, W), lambda b: (b, 0, 0)),
        pl.BlockSpec((1, T, 1), lambda b: (b, 0, 0)),
        full((1, W)), full((1, W)),
        full((W, FF)), full((1, FF)),
        full((FF, W)), full((1, W)),
        full((W, CD)), full((1, CD)),
        full((CD, W)), full((1, W)),
        full((KC, CD)),
        full((T, W)),
        full((1, W)), full((1, W)),
        full((W, W)), full((W, W)), full((W, W)), full((W, W)),
        full((1, W)), full((1, W)),
        full((W, FF)), full((1, FF)),
        full((FF, W)), full((1, W)),
        full((1, W)), full((1, W)),
    ]
    out_specs = [
        pl.BlockSpec((1, T, W), lambda b: (b, 0, 0)),
        pl.BlockSpec((1, T, 1), lambda b: (b, 0, 0)),
        pl.BlockSpec((1, 1, W), lambda b: (b, 0, 0)),
    ]
    out, idx3, commit = pl.pallas_call(
        _fused_kernel,
        grid=(B,),
        in_specs=in_specs,
        out_specs=out_specs,
        out_shape=[
            jax.ShapeDtypeStruct((B, T, W), jnp.float32),
            jax.ShapeDtypeStruct((B, T, 1), jnp.int32),
            jax.ShapeDtypeStruct((B, 1, W), jnp.float32),
        ],
        compiler_params=pltpu.CompilerParams(
            dimension_semantics=("parallel",)),
    )(embs, mask3, row(mlp_ln_g), row(mlp_ln_b), mlp_w1, row(mlp_b1),
      mlp_w2, row(mlp_b2), proj_in_w, row(proj_in_b), proj_out_w,
      row(proj_out_b), codebook, pos_emb, row(ln1_g), row(ln1_b),
      wq, wk, wv, wo, row(ln2_g), row(ln2_b),
      ffn_w1, row(ffn_b1),
      ffn_w2, row(ffn_b2), row(lnp_g), row(lnp_b))
    idx = idx3.reshape(B, T)
    commit_loss = jnp.sum(commit[:, 0, 0]) / (B * T * CD)
    return out, idx, commit_loss
